# baseline (speedup 1.0000x reference)
import jax
import jax.numpy as jnp
from jax.experimental import pallas as pl
from jax.experimental.pallas import tpu as pltpu

HIDDEN = 2048
INTER = 1408
IBG = 128
NBG = INTER // IBG
HBD = 256
NBD = HIDDEN // HBD


def _moe_body(idx_ref, w_ref, x_ref, gate_ref, up_ref, down_ref, out_ref, inter_ref):
    e = pl.program_id(0)
    s = pl.program_id(1)

    dn = (((1,), (1,)), ((), ()))

    @pl.when(s < NBG)
    def _gate_up():
        x = x_ref[...]
        g = gate_ref[0]
        u = up_ref[0]
        gate_out = jax.lax.dot_general(x, g, dn, preferred_element_type=jnp.float32)
        up_out = jax.lax.dot_general(x, u, dn, preferred_element_type=jnp.float32)
        inter = jax.nn.silu(gate_out) * up_out * w_ref[e]
        inter_ref[0, pl.ds(s * IBG, IBG)] = inter[0]

    @pl.when(jnp.logical_and(e == 0, s == NBG))
    def _init():
        out_ref[...] = jnp.zeros_like(out_ref)

    @pl.when(s >= NBG)
    def _down():
        h = s - NBG
        d = down_ref[0]
        inter = inter_ref[...]
        partial = jax.lax.dot_general(inter, d, dn, preferred_element_type=jnp.float32)
        out_ref[0, pl.ds(h * HBD, HBD)] += partial[0]


@jax.jit
def _run(x_flat, topk_idx, topk_weights, gate_proj_all, up_proj_all, down_proj_all):
    grid_spec = pltpu.PrefetchScalarGridSpec(
        num_scalar_prefetch=2,
        grid=(topk_idx.shape[0], NBG + NBD),
        in_specs=[
            pl.BlockSpec((1, HIDDEN), lambda e, s, idx, w: (0, 0)),
            pl.BlockSpec((1, IBG, HIDDEN),
                         lambda e, s, idx, w: (idx[e], jnp.minimum(s, NBG - 1), 0)),
            pl.BlockSpec((1, IBG, HIDDEN),
                         lambda e, s, idx, w: (idx[e], jnp.minimum(s, NBG - 1), 0)),
            pl.BlockSpec((1, HBD, INTER),
                         lambda e, s, idx, w: (idx[e], jnp.maximum(s - NBG, 0), 0)),
        ],
        out_specs=pl.BlockSpec((1, HIDDEN), lambda e, s, idx, w: (0, 0)),
        scratch_shapes=[pltpu.VMEM((1, INTER), jnp.float32)],
    )
    return pl.pallas_call(
        _moe_body,
        grid_spec=grid_spec,
        out_shape=jax.ShapeDtypeStruct((1, HIDDEN), jnp.float32),
        compiler_params=pltpu.CompilerParams(
            dimension_semantics=("arbitrary", "arbitrary"),
        ),
    )(topk_idx, topk_weights, x_flat, gate_proj_all, up_proj_all, down_proj_all)


def kernel(x_bc1t, topk_idx, topk_weights, gate_proj_all, up_proj_all, down_proj_all):
    x_flat = x_bc1t.reshape(1, HIDDEN)
    out = _run(x_flat, topk_idx.astype(jnp.int32), topk_weights,
               gate_proj_all, up_proj_all, down_proj_all)
    return out.reshape(1, HIDDEN, 1, 1)


# speedup vs baseline: 1.5294x; 1.5294x over previous
import jax
import jax.numpy as jnp
from jax.experimental import pallas as pl
from jax.experimental.pallas import tpu as pltpu

HIDDEN = 2048
INTER = 1408
IB = 256
NB = -(-INTER // IB)


def _moe_body(idx_ref, w_ref, x_ref, gate_ref, up_ref, down_ref, out_ref):
    e = pl.program_id(0)
    ib = pl.program_id(1)

    @pl.when(jnp.logical_and(e == 0, ib == 0))
    def _init():
        out_ref[...] = jnp.zeros_like(out_ref)

    x = x_ref[...]
    g = gate_ref[0]
    u = up_ref[0]
    d = down_ref[0]

    dn = (((1,), (1,)), ((), ()))
    gate_out = jax.lax.dot_general(x, g, dn, preferred_element_type=jnp.float32)
    up_out = jax.lax.dot_general(x, u, dn, preferred_element_type=jnp.float32)
    inter = jax.nn.silu(gate_out) * up_out * w_ref[e]
    col = jax.lax.broadcasted_iota(jnp.int32, (1, IB), 1) + ib * IB
    inter = jnp.where(col < INTER, inter, 0.0)
    partial = jax.lax.dot_general(inter, d, dn, preferred_element_type=jnp.float32)
    out_ref[...] += partial


@jax.jit
def _run(x_flat, topk_idx, topk_weights, gate_proj_all, up_proj_all, down_proj_all):
    grid_spec = pltpu.PrefetchScalarGridSpec(
        num_scalar_prefetch=2,
        grid=(topk_idx.shape[0], NB),
        in_specs=[
            pl.BlockSpec((1, HIDDEN), lambda e, ib, idx, w: (0, 0)),
            pl.BlockSpec((1, IB, HIDDEN), lambda e, ib, idx, w: (idx[e], ib, 0)),
            pl.BlockSpec((1, IB, HIDDEN), lambda e, ib, idx, w: (idx[e], ib, 0)),
            pl.BlockSpec((1, HIDDEN, IB), lambda e, ib, idx, w: (idx[e], 0, ib)),
        ],
        out_specs=pl.BlockSpec((1, HIDDEN), lambda e, ib, idx, w: (0, 0)),
    )
    return pl.pallas_call(
        _moe_body,
        grid_spec=grid_spec,
        out_shape=jax.ShapeDtypeStruct((1, HIDDEN), jnp.float32),
        compiler_params=pltpu.CompilerParams(
            dimension_semantics=("arbitrary", "arbitrary"),
        ),
    )(topk_idx, topk_weights, x_flat, gate_proj_all, up_proj_all, down_proj_all)


def kernel(x_bc1t, topk_idx, topk_weights, gate_proj_all, up_proj_all, down_proj_all):
    x_flat = x_bc1t.reshape(1, HIDDEN)
    out = _run(x_flat, topk_idx.astype(jnp.int32), topk_weights,
               gate_proj_all, up_proj_all, down_proj_all)
    return out.reshape(1, HIDDEN, 1, 1)


# speedup vs baseline: 1.7146x; 1.1211x over previous
import jax
import jax.numpy as jnp
from jax.experimental import pallas as pl
from jax.experimental.pallas import tpu as pltpu

HIDDEN = 2048
INTER = 1408
IB = 512
NB = -(-INTER // IB)


def _moe_body(idx_ref, w_ref, x_ref, gate_ref, up_ref, down_ref, out_ref):
    e = pl.program_id(0)
    ib = pl.program_id(1)

    @pl.when(jnp.logical_and(e == 0, ib == 0))
    def _init():
        out_ref[...] = jnp.zeros_like(out_ref)

    x = x_ref[...]
    g = gate_ref[0]
    u = up_ref[0]
    d = down_ref[0]

    dn = (((1,), (1,)), ((), ()))
    gate_out = jax.lax.dot_general(x, g, dn, preferred_element_type=jnp.float32)
    up_out = jax.lax.dot_general(x, u, dn, preferred_element_type=jnp.float32)
    inter = jax.nn.silu(gate_out) * up_out * w_ref[e]
    col = jax.lax.broadcasted_iota(jnp.int32, (1, IB), 1) + ib * IB
    inter = jnp.where(col < INTER, inter, 0.0)
    partial = jax.lax.dot_general(inter, d, dn, preferred_element_type=jnp.float32)
    out_ref[...] += partial


@jax.jit
def _run(x_flat, topk_idx, topk_weights, gate_proj_all, up_proj_all, down_proj_all):
    grid_spec = pltpu.PrefetchScalarGridSpec(
        num_scalar_prefetch=2,
        grid=(topk_idx.shape[0], NB),
        in_specs=[
            pl.BlockSpec((1, HIDDEN), lambda e, ib, idx, w: (0, 0)),
            pl.BlockSpec((1, IB, HIDDEN), lambda e, ib, idx, w: (idx[e], ib, 0)),
            pl.BlockSpec((1, IB, HIDDEN), lambda e, ib, idx, w: (idx[e], ib, 0)),
            pl.BlockSpec((1, HIDDEN, IB), lambda e, ib, idx, w: (idx[e], 0, ib)),
        ],
        out_specs=pl.BlockSpec((1, HIDDEN), lambda e, ib, idx, w: (0, 0)),
    )
    return pl.pallas_call(
        _moe_body,
        grid_spec=grid_spec,
        out_shape=jax.ShapeDtypeStruct((1, HIDDEN), jnp.float32),
        compiler_params=pltpu.CompilerParams(
            dimension_semantics=("arbitrary", "arbitrary"),
        ),
    )(topk_idx, topk_weights, x_flat, gate_proj_all, up_proj_all, down_proj_all)


def kernel(x_bc1t, topk_idx, topk_weights, gate_proj_all, up_proj_all, down_proj_all):
    x_flat = x_bc1t.reshape(1, HIDDEN)
    out = _run(x_flat, topk_idx.astype(jnp.int32), topk_weights,
               gate_proj_all, up_proj_all, down_proj_all)
    return out.reshape(1, HIDDEN, 1, 1)
